# baseline (speedup 1.0000x reference)
import functools

import jax
import jax.numpy as jnp
from jax import lax
from jax.experimental import pallas as pl
from jax.experimental.pallas import tpu as pltpu
from jax.experimental.pallas import tpu_sc as plsc

_LANES = 16
_ROW_BLOCK = 64


def _sc_scatter_row0(x, idx, val):
    cols = x.shape[1]
    n = idx.shape[0]
    mesh = plsc.VectorSubcoreMesh(core_axis_name="c", subcore_axis_name="s")

    @functools.partial(
        pl.kernel,
        out_type=jax.ShapeDtypeStruct((cols,), x.dtype),
        mesh=mesh,
        scratch_types=[
            pltpu.VMEM((cols,), x.dtype),
            pltpu.VMEM((n,), jnp.int32),
            pltpu.VMEM((n,), x.dtype),
            pltpu.SemaphoreType.DMA,
        ],
        compiler_params=pltpu.CompilerParams(needs_layout_passes=False),
    )
    def k(x_hbm, idx_hbm, val_hbm, o_hbm, row_v, idx_v, val_v, sem):
        @pl.when((lax.axis_index("c") == 0) & (lax.axis_index("s") == 0))
        def _():
            pltpu.async_copy(x_hbm.at[0], row_v, sem).wait()
            pltpu.sync_copy(idx_hbm, idx_v)
            pltpu.sync_copy(val_hbm, val_v)
            for j in range(n // _LANES):
                iv = idx_v[pl.ds(j * _LANES, _LANES)]
                vv = val_v[pl.ds(j * _LANES, _LANES)]
                plsc.store_scatter(row_v, [iv], vv)
            pltpu.sync_copy(row_v, o_hbm)

    return k(x, idx, val)


def _tc_copy_merge(x, row0):
    rows, cols = x.shape
    grid = rows // _ROW_BLOCK

    def body(x_ref, r0_ref, o_ref):
        o_ref[...] = x_ref[...]

        @pl.when(pl.program_id(0) == 0)
        def _():
            o_ref[0:1, :] = r0_ref[...]

    return pl.pallas_call(
        body,
        grid=(grid,),
        in_specs=[
            pl.BlockSpec((_ROW_BLOCK, cols), lambda i: (i, 0)),
            pl.BlockSpec((1, cols), lambda i: (0, 0)),
        ],
        out_specs=pl.BlockSpec((_ROW_BLOCK, cols), lambda i: (i, 0)),
        out_shape=jax.ShapeDtypeStruct((rows, cols), x.dtype),
        compiler_params=pltpu.CompilerParams(
            dimension_semantics=("arbitrary",)
        ),
    )(x, row0.reshape(1, cols))


def kernel(x, changed_genes, change_values):
    idx = changed_genes.astype(jnp.int32)
    n = idx.shape[0]
    pad = (-n) % _LANES
    if pad:
        idx = jnp.concatenate([idx, jnp.broadcast_to(idx[-1:], (pad,))])
        change_values = jnp.concatenate(
            [change_values, jnp.broadcast_to(change_values[-1:], (pad,))]
        )
    row0 = _sc_scatter_row0(x, idx, change_values)
    return _tc_copy_merge(x, row0)
